# Initial kernel scaffold; baseline (speedup 1.0000x reference)
import jax
import jax.numpy as jnp
from jax.experimental import pallas as pl


def kernel(x_center, x_neighbors, neighbor_idx, rel_pos, W_pos, bn_gamma, bn_beta, W_mix, b_mix, W_val, b_val):
    raise NotImplementedError("write your pallas kernel here")


# speedup vs baseline: 5.1308x; 5.1308x over previous
import functools

import jax
import jax.numpy as jnp
from jax import lax
from jax.experimental import pallas as pl
from jax.experimental.pallas import tpu as pltpu
from jax.experimental.pallas import tpu_sc as plsc

N_NODES = 10000
NSAMPLE = 32
IN_PLANES = 128
SHARE_PLANES = 8
MID = IN_PLANES // SHARE_PLANES
EPS = 1e-5
E = N_NODES * NSAMPLE

BE = 4000
GRID_E = E // BE

NC = 2
NS = 16
NW = NC * NS
EPW = E // NW
CH = 80
KCH = EPW // CH
RPW = 640
NPAD = NS * RPW

BN = 1000
GRID_N = N_NODES // BN


def _stats_body(rp_ref, w4_ref, s_ref, q_ref):
    g = pl.program_id(0)
    h = jnp.dot(rp_ref[...], w4_ref[...], preferred_element_type=jnp.float32)

    @pl.when(g == 0)
    def _():
        s_ref[...] = jnp.zeros_like(s_ref)
        q_ref[...] = jnp.zeros_like(q_ref)

    s_ref[...] += jnp.sum(h, axis=0, keepdims=True)
    q_ref[...] += jnp.sum(h * h, axis=0, keepdims=True)


def _dense_body(ss_ref, sq_ref, gam_ref, bet_ref, rp_ref, xn_ref, w4_ref,
                wm1_ref, wcat_ref, bcat_ref, out_ref):
    inv_e = jnp.float32(1.0 / E)
    mean = ss_ref[...] * inv_e
    var = sq_ref[...] * inv_e - mean * mean
    scale = gam_ref[...] * lax.rsqrt(var + EPS)
    shift = bet_ref[...] - mean * scale
    h = jnp.dot(rp_ref[...], w4_ref[...], preferred_element_type=jnp.float32)
    pos = jnp.maximum(h * scale + shift, 0.0)
    t = jnp.dot(xn_ref[...], wcat_ref[...],
                preferred_element_type=jnp.float32) + bcat_ref[...]
    mpos = jnp.dot(pos, wm1_ref[...], preferred_element_type=jnp.float32)
    m = jnp.maximum(t[:, :MID] + mpos, 0.0)
    v = jnp.maximum(t[:, MID:], 0.0)
    ex = jnp.exp(m)
    out_ref[...] = jnp.concatenate([ex, v * ex], axis=1)


def _scatter_body(idx_hbm, contrib_hbm, zero_hbm, out_hbm, idx_v, c32, c128,
                  acc):
    c = lax.axis_index("c")
    s = lax.axis_index("s")
    wid = c * NS + s

    pltpu.sync_copy(zero_hbm, acc.at[pl.ds(s * RPW, RPW)])
    pltpu.sync_copy(zero_hbm.at[pl.ds(0, CH)], c128)
    plsc.subcore_barrier()

    pltpu.sync_copy(idx_hbm.at[wid], idx_v)

    def chunk(j, carry):
        pltpu.sync_copy(contrib_hbm.at[wid, j], c32)

        def repack(r, carry2):
            c128[r, pl.ds(0, 16)] = c32[r, pl.ds(0, 16)]
            c128[r, pl.ds(16, 16)] = c32[r, pl.ds(16, 16)]
            return carry2

        lax.fori_loop(0, CH, repack, 0, unroll=4)
        pltpu.sync_copy(c128, acc.at[idx_v.at[j]], add=True)
        return carry

    lax.fori_loop(0, KCH, chunk, 0, unroll=False)

    plsc.subcore_barrier()
    pltpu.sync_copy(acc.at[pl.ds(s * RPW, RPW)],
                    out_hbm.at[c, pl.ds(s * RPW, RPW)])


def _combine_body(pa_ref, pb_ref, xc_ref, out_ref):
    p = pa_ref[...] + pb_ref[...]
    s0 = p[:, :MID]
    s1 = p[:, MID:]
    nonempty = s0 > 0.0
    agg = jnp.where(nonempty, s1 / jnp.where(nonempty, s0, 1.0), 0.0)
    out_ref[...] = xc_ref[...] + jnp.concatenate([agg] * SHARE_PLANES, axis=1)


def kernel(x_center, x_neighbors, neighbor_idx, rel_pos, W_pos, bn_gamma,
           bn_beta, W_mix, b_mix, W_val, b_val):
    f32 = jnp.float32
    rp4 = jnp.pad(rel_pos.reshape(E, 3), ((0, 0), (0, 1)))
    w4 = jnp.pad(W_pos, ((0, 1), (0, 0)))
    xn = x_neighbors.reshape(E, IN_PLANES)
    wm1 = W_mix[:IN_PLANES]
    wcat = jnp.concatenate([W_mix[IN_PLANES:], W_val], axis=1)
    bcat = jnp.concatenate([b_mix, b_val]).reshape(1, 2 * MID)
    gam = bn_gamma.reshape(1, IN_PLANES)
    bet = bn_beta.reshape(1, IN_PLANES)

    ss, sq = pl.pallas_call(
        _stats_body,
        grid=(GRID_E,),
        in_specs=[
            pl.BlockSpec((BE, 4), lambda g: (g, 0)),
            pl.BlockSpec((4, IN_PLANES), lambda g: (0, 0)),
        ],
        out_specs=[
            pl.BlockSpec((1, IN_PLANES), lambda g: (0, 0)),
            pl.BlockSpec((1, IN_PLANES), lambda g: (0, 0)),
        ],
        out_shape=[
            jax.ShapeDtypeStruct((1, IN_PLANES), f32),
            jax.ShapeDtypeStruct((1, IN_PLANES), f32),
        ],
        compiler_params=pltpu.CompilerParams(
            dimension_semantics=("arbitrary",)),
    )(rp4, w4)

    contrib = pl.pallas_call(
        _dense_body,
        grid=(GRID_E,),
        in_specs=[
            pl.BlockSpec((1, IN_PLANES), lambda g: (0, 0)),
            pl.BlockSpec((1, IN_PLANES), lambda g: (0, 0)),
            pl.BlockSpec((1, IN_PLANES), lambda g: (0, 0)),
            pl.BlockSpec((1, IN_PLANES), lambda g: (0, 0)),
            pl.BlockSpec((BE, 4), lambda g: (g, 0)),
            pl.BlockSpec((BE, IN_PLANES), lambda g: (g, 0)),
            pl.BlockSpec((4, IN_PLANES), lambda g: (0, 0)),
            pl.BlockSpec((IN_PLANES, MID), lambda g: (0, 0)),
            pl.BlockSpec((IN_PLANES, 2 * MID), lambda g: (0, 0)),
            pl.BlockSpec((1, 2 * MID), lambda g: (0, 0)),
        ],
        out_specs=pl.BlockSpec((BE, 2 * MID), lambda g: (g, 0)),
        out_shape=jax.ShapeDtypeStruct((E, 2 * MID), f32),
        compiler_params=pltpu.CompilerParams(
            dimension_semantics=("arbitrary",)),
    )(ss, sq, gam, bet, rp4, xn, w4, wm1, wcat, bcat)

    idx3 = neighbor_idx.reshape(NW, KCH, CH).astype(jnp.int32)
    c4 = contrib.reshape(NW, KCH, CH, 2 * MID)
    zeros = jnp.zeros((RPW, 128), f32)

    scatter = functools.partial(
        pl.kernel,
        out_type=jax.ShapeDtypeStruct((NC, NPAD, 128), f32),
        mesh=plsc.VectorSubcoreMesh(core_axis_name="c", subcore_axis_name="s",
                                    num_cores=NC, num_subcores=NS),
        scratch_types=[
            pltpu.VMEM((KCH, CH), jnp.int32),
            pltpu.VMEM((CH, 2 * MID), f32),
            pltpu.VMEM((CH, 128), f32),
            pltpu.VMEM_SHARED((NPAD, 128), f32),
        ],
    )(_scatter_body)
    parts = scatter(idx3, c4, zeros)
    pa = parts[0, :N_NODES, :2 * MID]
    pb = parts[1, :N_NODES, :2 * MID]

    out = pl.pallas_call(
        _combine_body,
        grid=(GRID_N,),
        in_specs=[
            pl.BlockSpec((BN, 2 * MID), lambda g: (g, 0)),
            pl.BlockSpec((BN, 2 * MID), lambda g: (g, 0)),
            pl.BlockSpec((BN, IN_PLANES), lambda g: (g, 0)),
        ],
        out_specs=pl.BlockSpec((BN, IN_PLANES), lambda g: (g, 0)),
        out_shape=jax.ShapeDtypeStruct((N_NODES, IN_PLANES), f32),
        compiler_params=pltpu.CompilerParams(
            dimension_semantics=("arbitrary",)),
    )(pa, pb, x_center)
    return out


# speedup vs baseline: 6.6777x; 1.3015x over previous
import functools

import jax
import jax.numpy as jnp
from jax import lax
from jax.experimental import pallas as pl
from jax.experimental.pallas import tpu as pltpu
from jax.experimental.pallas import tpu_sc as plsc

N_NODES = 10000
NSAMPLE = 32
IN_PLANES = 128
SHARE_PLANES = 8
MID = IN_PLANES // SHARE_PLANES
EPS = 1e-5
E = N_NODES * NSAMPLE

BE = 8000
GRID_E = E // BE

NC = 2
NS = 16
NW = NC * NS
EPW = E // NW
CH = 80
KCH = EPW // CH
RPW = 640
NPAD = NS * RPW

BN = 1000
GRID_N = N_NODES // BN


def _stats_body(rp_ref, w4_ref, s_ref, q_ref):
    g = pl.program_id(0)
    h = jnp.dot(rp_ref[...], w4_ref[...], preferred_element_type=jnp.float32)

    @pl.when(g == 0)
    def _():
        s_ref[...] = jnp.zeros_like(s_ref)
        q_ref[...] = jnp.zeros_like(q_ref)

    s_ref[...] += jnp.sum(h, axis=0, keepdims=True)
    q_ref[...] += jnp.sum(h * h, axis=0, keepdims=True)


def _dense_body(ss_ref, sq_ref, gam_ref, bet_ref, rp_ref, xn_ref, w4_ref,
                wm1_ref, wcat_ref, bcat_ref, out_ref):
    inv_e = jnp.float32(1.0 / E)
    mean = ss_ref[...] * inv_e
    var = sq_ref[...] * inv_e - mean * mean
    scale = gam_ref[...] * lax.rsqrt(var + EPS)
    shift = bet_ref[...] - mean * scale
    h = jnp.dot(rp_ref[...], w4_ref[...], preferred_element_type=jnp.float32)
    pos = jnp.maximum(h * scale + shift, 0.0)
    t = jnp.dot(xn_ref[...], wcat_ref[...],
                preferred_element_type=jnp.float32) + bcat_ref[...]
    mpos = jnp.dot(pos, wm1_ref[...], preferred_element_type=jnp.float32)
    m = jnp.maximum(t[:, :MID] + mpos, 0.0)
    v = jnp.maximum(t[:, MID:], 0.0)
    ex = jnp.exp(m)
    out_ref[...] = jnp.concatenate([ex, v * ex], axis=1)


def _scatter_body(idx_hbm, contrib_hbm, zero_hbm, out_hbm, idx_v, c32a, c32b,
                  c128, acc, sema, semb):
    c = lax.axis_index("c")
    s = lax.axis_index("s")
    wid = c * NS + s

    pltpu.sync_copy(zero_hbm, acc.at[pl.ds(s * RPW, RPW)])
    pltpu.sync_copy(zero_hbm.at[pl.ds(0, CH)], c128)
    plsc.subcore_barrier()

    pltpu.sync_copy(idx_hbm.at[wid], idx_v)

    def repack_scatter(j, c32):
        def repack(r, carry2):
            c128[r, pl.ds(0, 16)] = c32[r, pl.ds(0, 16)]
            c128[r, pl.ds(16, 16)] = c32[r, pl.ds(16, 16)]
            return carry2

        lax.fori_loop(0, CH, repack, 0, unroll=8)
        pltpu.sync_copy(c128, acc.at[idx_v.at[j]], add=True)

    NP2 = (KCH - 1) // 2
    pltpu.async_copy(contrib_hbm.at[wid, 1], c32a, sema)
    pltpu.sync_copy(contrib_hbm.at[wid, 0], c32b)
    repack_scatter(0, c32b)

    def pair(p, carry):
        j0 = 2 * p + 1
        pltpu.async_copy(contrib_hbm.at[wid, j0 + 1], c32b, semb)
        pltpu.make_async_copy(contrib_hbm.at[wid, j0], c32a, sema).wait()
        repack_scatter(j0, c32a)

        @pl.when(p + 1 < NP2)
        def _():
            pltpu.async_copy(contrib_hbm.at[wid, j0 + 2], c32a, sema)

        pltpu.make_async_copy(contrib_hbm.at[wid, j0 + 1], c32b, semb).wait()
        repack_scatter(j0 + 1, c32b)
        return carry

    lax.fori_loop(0, NP2, pair, 0, unroll=False)

    plsc.subcore_barrier()
    pltpu.sync_copy(acc.at[pl.ds(s * RPW, RPW)],
                    out_hbm.at[c, pl.ds(s * RPW, RPW)])


def _combine_body(pa_ref, pb_ref, xc_ref, out_ref):
    p = pa_ref[...] + pb_ref[...]
    s0 = p[:, :MID]
    s1 = p[:, MID:]
    nonempty = s0 > 0.0
    agg = jnp.where(nonempty, s1 / jnp.where(nonempty, s0, 1.0), 0.0)
    out_ref[...] = xc_ref[...] + jnp.concatenate([agg] * SHARE_PLANES, axis=1)


def kernel(x_center, x_neighbors, neighbor_idx, rel_pos, W_pos, bn_gamma,
           bn_beta, W_mix, b_mix, W_val, b_val):
    f32 = jnp.float32
    rp4 = jnp.pad(rel_pos.reshape(E, 3), ((0, 0), (0, 1)))
    w4 = jnp.pad(W_pos, ((0, 1), (0, 0)))
    xn = x_neighbors.reshape(E, IN_PLANES)
    wm1 = W_mix[:IN_PLANES]
    wcat = jnp.concatenate([W_mix[IN_PLANES:], W_val], axis=1)
    bcat = jnp.concatenate([b_mix, b_val]).reshape(1, 2 * MID)
    gam = bn_gamma.reshape(1, IN_PLANES)
    bet = bn_beta.reshape(1, IN_PLANES)

    ss, sq = pl.pallas_call(
        _stats_body,
        grid=(GRID_E,),
        in_specs=[
            pl.BlockSpec((BE, 4), lambda g: (g, 0)),
            pl.BlockSpec((4, IN_PLANES), lambda g: (0, 0)),
        ],
        out_specs=[
            pl.BlockSpec((1, IN_PLANES), lambda g: (0, 0)),
            pl.BlockSpec((1, IN_PLANES), lambda g: (0, 0)),
        ],
        out_shape=[
            jax.ShapeDtypeStruct((1, IN_PLANES), f32),
            jax.ShapeDtypeStruct((1, IN_PLANES), f32),
        ],
        compiler_params=pltpu.CompilerParams(
            dimension_semantics=("arbitrary",)),
    )(rp4, w4)

    contrib = pl.pallas_call(
        _dense_body,
        grid=(GRID_E,),
        in_specs=[
            pl.BlockSpec((1, IN_PLANES), lambda g: (0, 0)),
            pl.BlockSpec((1, IN_PLANES), lambda g: (0, 0)),
            pl.BlockSpec((1, IN_PLANES), lambda g: (0, 0)),
            pl.BlockSpec((1, IN_PLANES), lambda g: (0, 0)),
            pl.BlockSpec((BE, 4), lambda g: (g, 0)),
            pl.BlockSpec((BE, IN_PLANES), lambda g: (g, 0)),
            pl.BlockSpec((4, IN_PLANES), lambda g: (0, 0)),
            pl.BlockSpec((IN_PLANES, MID), lambda g: (0, 0)),
            pl.BlockSpec((IN_PLANES, 2 * MID), lambda g: (0, 0)),
            pl.BlockSpec((1, 2 * MID), lambda g: (0, 0)),
        ],
        out_specs=pl.BlockSpec((BE, 2 * MID), lambda g: (g, 0)),
        out_shape=jax.ShapeDtypeStruct((E, 2 * MID), f32),
        compiler_params=pltpu.CompilerParams(
            dimension_semantics=("arbitrary",)),
    )(ss, sq, gam, bet, rp4, xn, w4, wm1, wcat, bcat)

    idx3 = neighbor_idx.reshape(NW, KCH, CH).astype(jnp.int32)
    c4 = contrib.reshape(NW, KCH, CH, 2 * MID)
    zeros = jnp.zeros((RPW, 128), f32)

    scatter = functools.partial(
        pl.kernel,
        out_type=jax.ShapeDtypeStruct((NC, NPAD, 128), f32),
        mesh=plsc.VectorSubcoreMesh(core_axis_name="c", subcore_axis_name="s",
                                    num_cores=NC, num_subcores=NS),
        scratch_types=[
            pltpu.VMEM((KCH, CH), jnp.int32),
            pltpu.VMEM((CH, 2 * MID), f32),
            pltpu.VMEM((CH, 2 * MID), f32),
            pltpu.VMEM((CH, 128), f32),
            pltpu.VMEM_SHARED((NPAD, 128), f32),
            pltpu.SemaphoreType.DMA,
            pltpu.SemaphoreType.DMA,
        ],
    )(_scatter_body)
    parts = scatter(idx3, c4, zeros)
    pa = parts[0, :N_NODES, :2 * MID]
    pb = parts[1, :N_NODES, :2 * MID]

    out = pl.pallas_call(
        _combine_body,
        grid=(GRID_N,),
        in_specs=[
            pl.BlockSpec((BN, 2 * MID), lambda g: (g, 0)),
            pl.BlockSpec((BN, 2 * MID), lambda g: (g, 0)),
            pl.BlockSpec((BN, IN_PLANES), lambda g: (g, 0)),
        ],
        out_specs=pl.BlockSpec((BN, IN_PLANES), lambda g: (g, 0)),
        out_shape=jax.ShapeDtypeStruct((N_NODES, IN_PLANES), f32),
        compiler_params=pltpu.CompilerParams(
            dimension_semantics=("arbitrary",)),
    )(pa, pb, x_center)
    return out


# speedup vs baseline: 9.4588x; 1.4165x over previous
import functools

import jax
import jax.numpy as jnp
from jax import lax
from jax.experimental import pallas as pl
from jax.experimental.pallas import tpu as pltpu
from jax.experimental.pallas import tpu_sc as plsc

N_NODES = 10000
NSAMPLE = 32
IN_PLANES = 128
SHARE_PLANES = 8
MID = IN_PLANES // SHARE_PLANES
EPS = 1e-5
E = N_NODES * NSAMPLE

BE = 6400
GRID_E = E // BE

NC = 2
NS = 16
NW = NC * NS
EPW = E // NW
CH = 80
KCH = EPW // CH
RPW = 640
NPAD = NS * RPW

BN = 1000
GRID_N = N_NODES // BN


def _stats_body(rp_ref, w4_ref, s_ref, q_ref):
    g = pl.program_id(0)
    h = lax.dot_general(rp_ref[...], w4_ref[...],
                        dimension_numbers=(((0,), (0,)), ((), ())),
                        preferred_element_type=jnp.float32)

    @pl.when(g == 0)
    def _():
        s_ref[...] = jnp.zeros_like(s_ref)
        q_ref[...] = jnp.zeros_like(q_ref)

    s_ref[...] += jnp.sum(h, axis=0, keepdims=True)
    q_ref[...] += jnp.sum(h * h, axis=0, keepdims=True)


def _dense_body(ss_ref, sq_ref, gam_ref, bet_ref, rp_ref, xn_ref, w4_ref,
                wm1_ref, wcat_ref, bcat_ref, out_ref):
    inv_e = jnp.float32(1.0 / E)
    mean = ss_ref[...] * inv_e
    var = sq_ref[...] * inv_e - mean * mean
    scale = gam_ref[...] * lax.rsqrt(var + EPS)
    shift = bet_ref[...] - mean * scale
    h = lax.dot_general(rp_ref[...], w4_ref[...],
                        dimension_numbers=(((0,), (0,)), ((), ())),
                        preferred_element_type=jnp.float32)
    pos = jnp.maximum(h * scale + shift, 0.0)
    t = jnp.dot(xn_ref[...], wcat_ref[...],
                preferred_element_type=jnp.float32) + bcat_ref[...]
    mpos = jnp.dot(pos, wm1_ref[...], preferred_element_type=jnp.float32)
    m = jnp.maximum(t[:, :MID] + mpos, 0.0)
    v = jnp.maximum(t[:, MID:], 0.0)
    ex = jnp.exp(m)
    out_ref[...] = jnp.concatenate([ex, v * ex], axis=1)


def _scatter_body(idx_hbm, contrib_hbm, zero_hbm, out_hbm, idx_v, c32a, c32b,
                  c128, acc, sema, semb):
    c = lax.axis_index("c")
    s = lax.axis_index("s")
    wid = c * NS + s

    pltpu.sync_copy(zero_hbm, acc.at[pl.ds(s * RPW, RPW)])
    pltpu.sync_copy(zero_hbm.at[pl.ds(0, CH)], c128)
    plsc.subcore_barrier()

    pltpu.sync_copy(idx_hbm.at[wid], idx_v)

    def repack_scatter(j, c32):
        def repack(r, carry2):
            c128[r, pl.ds(0, 16)] = c32[r, pl.ds(0, 16)]
            c128[r, pl.ds(16, 16)] = c32[r, pl.ds(16, 16)]
            return carry2

        lax.fori_loop(0, CH, repack, 0, unroll=8)
        pltpu.sync_copy(c128, acc.at[idx_v.at[j]], add=True)

    NP2 = (KCH - 1) // 2
    pltpu.async_copy(contrib_hbm.at[wid, 1], c32a, sema)
    pltpu.sync_copy(contrib_hbm.at[wid, 0], c32b)
    repack_scatter(0, c32b)

    def pair(p, carry):
        j0 = 2 * p + 1
        pltpu.async_copy(contrib_hbm.at[wid, j0 + 1], c32b, semb)
        pltpu.make_async_copy(contrib_hbm.at[wid, j0], c32a, sema).wait()
        repack_scatter(j0, c32a)

        @pl.when(p + 1 < NP2)
        def _():
            pltpu.async_copy(contrib_hbm.at[wid, j0 + 2], c32a, sema)

        pltpu.make_async_copy(contrib_hbm.at[wid, j0 + 1], c32b, semb).wait()
        repack_scatter(j0 + 1, c32b)
        return carry

    lax.fori_loop(0, NP2, pair, 0, unroll=False)

    plsc.subcore_barrier()
    pltpu.sync_copy(acc.at[pl.ds(s * RPW, RPW)],
                    out_hbm.at[c, pl.ds(s * RPW, RPW)])


def _combine_body(parts_ref, xc_ref, out_ref):
    p = parts_ref[0] + parts_ref[1]
    s0 = p[:, :MID]
    s1 = p[:, MID:2 * MID]
    nonempty = s0 > 0.0
    agg = jnp.where(nonempty, s1 / jnp.where(nonempty, s0, 1.0), 0.0)
    out_ref[...] = xc_ref[...] + jnp.concatenate([agg] * SHARE_PLANES, axis=1)


def kernel(x_center, x_neighbors, neighbor_idx, rel_pos, W_pos, bn_gamma,
           bn_beta, W_mix, b_mix, W_val, b_val):
    f32 = jnp.float32
    rpt = rel_pos.reshape(E, 3).T
    xn = x_neighbors.reshape(E, IN_PLANES)
    wm1 = W_mix[:IN_PLANES]
    wcat = jnp.concatenate([W_mix[IN_PLANES:], W_val], axis=1)
    bcat = jnp.concatenate([b_mix, b_val]).reshape(1, 2 * MID)
    gam = bn_gamma.reshape(1, IN_PLANES)
    bet = bn_beta.reshape(1, IN_PLANES)

    ss, sq = pl.pallas_call(
        _stats_body,
        grid=(GRID_E,),
        in_specs=[
            pl.BlockSpec((3, BE), lambda g: (0, g)),
            pl.BlockSpec((3, IN_PLANES), lambda g: (0, 0)),
        ],
        out_specs=[
            pl.BlockSpec((1, IN_PLANES), lambda g: (0, 0)),
            pl.BlockSpec((1, IN_PLANES), lambda g: (0, 0)),
        ],
        out_shape=[
            jax.ShapeDtypeStruct((1, IN_PLANES), f32),
            jax.ShapeDtypeStruct((1, IN_PLANES), f32),
        ],
        compiler_params=pltpu.CompilerParams(
            dimension_semantics=("arbitrary",)),
    )(rpt, W_pos)

    contrib = pl.pallas_call(
        _dense_body,
        grid=(GRID_E,),
        in_specs=[
            pl.BlockSpec((1, IN_PLANES), lambda g: (0, 0)),
            pl.BlockSpec((1, IN_PLANES), lambda g: (0, 0)),
            pl.BlockSpec((1, IN_PLANES), lambda g: (0, 0)),
            pl.BlockSpec((1, IN_PLANES), lambda g: (0, 0)),
            pl.BlockSpec((3, BE), lambda g: (0, g)),
            pl.BlockSpec((BE, IN_PLANES), lambda g: (g, 0)),
            pl.BlockSpec((3, IN_PLANES), lambda g: (0, 0)),
            pl.BlockSpec((IN_PLANES, MID), lambda g: (0, 0)),
            pl.BlockSpec((IN_PLANES, 2 * MID), lambda g: (0, 0)),
            pl.BlockSpec((1, 2 * MID), lambda g: (0, 0)),
        ],
        out_specs=pl.BlockSpec((BE, 2 * MID), lambda g: (g, 0)),
        out_shape=jax.ShapeDtypeStruct((E, 2 * MID), f32),
        compiler_params=pltpu.CompilerParams(
            dimension_semantics=("arbitrary",)),
    )(ss, sq, gam, bet, rpt, xn, W_pos, wm1, wcat, bcat)

    idx3 = neighbor_idx.reshape(NW, KCH, CH).astype(jnp.int32)
    c4 = contrib.reshape(NW, KCH, CH, 2 * MID)
    zeros = jnp.zeros((RPW, 128), f32)

    scatter = functools.partial(
        pl.kernel,
        out_type=jax.ShapeDtypeStruct((NC, NPAD, 128), f32),
        mesh=plsc.VectorSubcoreMesh(core_axis_name="c", subcore_axis_name="s",
                                    num_cores=NC, num_subcores=NS),
        scratch_types=[
            pltpu.VMEM((KCH, CH), jnp.int32),
            pltpu.VMEM((CH, 2 * MID), f32),
            pltpu.VMEM((CH, 2 * MID), f32),
            pltpu.VMEM((CH, 128), f32),
            pltpu.VMEM_SHARED((NPAD, 128), f32),
            pltpu.SemaphoreType.DMA,
            pltpu.SemaphoreType.DMA,
        ],
    )(_scatter_body)
    parts = scatter(idx3, c4, zeros)

    out = pl.pallas_call(
        _combine_body,
        grid=(GRID_N,),
        in_specs=[
            pl.BlockSpec((NC, BN, 128), lambda g: (0, g, 0)),
            pl.BlockSpec((BN, IN_PLANES), lambda g: (g, 0)),
        ],
        out_specs=pl.BlockSpec((BN, IN_PLANES), lambda g: (g, 0)),
        out_shape=jax.ShapeDtypeStruct((N_NODES, IN_PLANES), f32),
        compiler_params=pltpu.CompilerParams(
            dimension_semantics=("arbitrary",)),
    )(parts, x_center)
    return out


# speedup vs baseline: 11.1674x; 1.1806x over previous
import functools

import jax
import jax.numpy as jnp
from jax import lax
from jax.experimental import pallas as pl
from jax.experimental.pallas import tpu as pltpu
from jax.experimental.pallas import tpu_sc as plsc

N_NODES = 10000
NSAMPLE = 32
IN_PLANES = 128
SHARE_PLANES = 8
MID = IN_PLANES // SHARE_PLANES
EPS = 1e-5
E = N_NODES * NSAMPLE

BE = 6400
GRID_E = E // BE

NC = 2
NS = 16
NW = NC * NS
EPW = E // NW
CH = 80
KCH = EPW // CH
RPW = 640
NPAD = NS * RPW

BN = 1000
GRID_N = N_NODES // BN


def _stats_body(rp_ref, s_ref, g_ref):
    g = pl.program_id(0)
    rp = rp_ref[...]
    gram = lax.dot_general(rp, rp, dimension_numbers=(((1,), (1,)), ((), ())),
                           preferred_element_type=jnp.float32)

    @pl.when(g == 0)
    def _():
        s_ref[...] = jnp.zeros_like(s_ref)
        g_ref[...] = jnp.zeros_like(g_ref)

    s_ref[...] += jnp.sum(rp, axis=1, keepdims=True)
    g_ref[...] += gram


def _dense_body(ss_ref, gg_ref, gam_ref, bet_ref, rp_ref, xn_ref, w4_ref,
                wm1_ref, wcat_ref, bcat_ref, out_ref):
    inv_e = jnp.float32(1.0 / E)
    w = w4_ref[...]
    mean = lax.dot_general(ss_ref[...], w,
                           dimension_numbers=(((0,), (0,)), ((), ())),
                           preferred_element_type=jnp.float32) * inv_e
    t = jnp.dot(gg_ref[...], w, preferred_element_type=jnp.float32)
    e2 = jnp.sum(w * t, axis=0, keepdims=True) * inv_e
    var = e2 - mean * mean
    scale = gam_ref[...] * lax.rsqrt(var + EPS)
    shift = bet_ref[...] - mean * scale
    h = lax.dot_general(rp_ref[...], w4_ref[...],
                        dimension_numbers=(((0,), (0,)), ((), ())),
                        preferred_element_type=jnp.float32)
    pos = jnp.maximum(h * scale + shift, 0.0)
    t = jnp.dot(xn_ref[...], wcat_ref[...],
                preferred_element_type=jnp.float32) + bcat_ref[...]
    mpos = jnp.dot(pos, wm1_ref[...], preferred_element_type=jnp.float32)
    m = jnp.maximum(t[:, :MID] + mpos, 0.0)
    v = jnp.maximum(t[:, MID:], 0.0)
    ex = jnp.exp(m)
    out_ref[...] = jnp.concatenate([ex, v * ex], axis=1)


def _scatter_body(idx_hbm, contrib_hbm, zero_hbm, out_hbm, idx_v, c32a, c32b,
                  c128, acc, sema, semb):
    c = lax.axis_index("c")
    s = lax.axis_index("s")
    wid = c * NS + s

    pltpu.sync_copy(zero_hbm, acc.at[pl.ds(s * RPW, RPW)])
    pltpu.sync_copy(zero_hbm.at[pl.ds(0, CH)], c128)
    plsc.subcore_barrier()

    pltpu.sync_copy(idx_hbm.at[wid], idx_v)

    def repack_scatter(j, c32):
        def repack(r, carry2):
            c128[r, pl.ds(0, 16)] = c32[r, pl.ds(0, 16)]
            c128[r, pl.ds(16, 16)] = c32[r, pl.ds(16, 16)]
            return carry2

        lax.fori_loop(0, CH, repack, 0, unroll=8)
        pltpu.sync_copy(c128, acc.at[idx_v.at[j]], add=True)

    NP2 = (KCH - 1) // 2
    pltpu.async_copy(contrib_hbm.at[wid, 1], c32a, sema)
    pltpu.sync_copy(contrib_hbm.at[wid, 0], c32b)
    repack_scatter(0, c32b)

    def pair(p, carry):
        j0 = 2 * p + 1
        pltpu.async_copy(contrib_hbm.at[wid, j0 + 1], c32b, semb)
        pltpu.make_async_copy(contrib_hbm.at[wid, j0], c32a, sema).wait()
        repack_scatter(j0, c32a)

        @pl.when(p + 1 < NP2)
        def _():
            pltpu.async_copy(contrib_hbm.at[wid, j0 + 2], c32a, sema)

        pltpu.make_async_copy(contrib_hbm.at[wid, j0 + 1], c32b, semb).wait()
        repack_scatter(j0 + 1, c32b)
        return carry

    lax.fori_loop(0, NP2, pair, 0, unroll=False)

    plsc.subcore_barrier()
    pltpu.sync_copy(acc.at[pl.ds(s * RPW, RPW)],
                    out_hbm.at[c, pl.ds(s * RPW, RPW)])


def _combine_body(parts_ref, xc_ref, out_ref):
    p = parts_ref[0] + parts_ref[1]
    s0 = p[:, :MID]
    s1 = p[:, MID:2 * MID]
    nonempty = s0 > 0.0
    agg = jnp.where(nonempty, s1 / jnp.where(nonempty, s0, 1.0), 0.0)
    out_ref[...] = xc_ref[...] + jnp.concatenate([agg] * SHARE_PLANES, axis=1)


def kernel(x_center, x_neighbors, neighbor_idx, rel_pos, W_pos, bn_gamma,
           bn_beta, W_mix, b_mix, W_val, b_val):
    f32 = jnp.float32
    rpt = rel_pos.reshape(E, 3).T
    xn = x_neighbors.reshape(E, IN_PLANES)
    wm1 = W_mix[:IN_PLANES]
    wcat = jnp.concatenate([W_mix[IN_PLANES:], W_val], axis=1)
    bcat = jnp.concatenate([b_mix, b_val]).reshape(1, 2 * MID)
    gam = bn_gamma.reshape(1, IN_PLANES)
    bet = bn_beta.reshape(1, IN_PLANES)

    ss, sq = pl.pallas_call(
        _stats_body,
        grid=(GRID_E,),
        in_specs=[
            pl.BlockSpec((3, BE), lambda g: (0, g)),
        ],
        out_specs=[
            pl.BlockSpec((3, 1), lambda g: (0, 0)),
            pl.BlockSpec((3, 3), lambda g: (0, 0)),
        ],
        out_shape=[
            jax.ShapeDtypeStruct((3, 1), f32),
            jax.ShapeDtypeStruct((3, 3), f32),
        ],
        compiler_params=pltpu.CompilerParams(
            dimension_semantics=("arbitrary",)),
    )(rpt)

    contrib = pl.pallas_call(
        _dense_body,
        grid=(GRID_E,),
        in_specs=[
            pl.BlockSpec((3, 1), lambda g: (0, 0)),
            pl.BlockSpec((3, 3), lambda g: (0, 0)),
            pl.BlockSpec((1, IN_PLANES), lambda g: (0, 0)),
            pl.BlockSpec((1, IN_PLANES), lambda g: (0, 0)),
            pl.BlockSpec((3, BE), lambda g: (0, g)),
            pl.BlockSpec((BE, IN_PLANES), lambda g: (g, 0)),
            pl.BlockSpec((3, IN_PLANES), lambda g: (0, 0)),
            pl.BlockSpec((IN_PLANES, MID), lambda g: (0, 0)),
            pl.BlockSpec((IN_PLANES, 2 * MID), lambda g: (0, 0)),
            pl.BlockSpec((1, 2 * MID), lambda g: (0, 0)),
        ],
        out_specs=pl.BlockSpec((BE, 2 * MID), lambda g: (g, 0)),
        out_shape=jax.ShapeDtypeStruct((E, 2 * MID), f32),
        compiler_params=pltpu.CompilerParams(
            dimension_semantics=("arbitrary",)),
    )(ss, sq, gam, bet, rpt, xn, W_pos, wm1, wcat, bcat)

    idx3 = neighbor_idx.reshape(NW, KCH, CH).astype(jnp.int32)
    c4 = contrib.reshape(NW, KCH, CH, 2 * MID)
    zeros = jnp.zeros((RPW, 128), f32)

    scatter = functools.partial(
        pl.kernel,
        out_type=jax.ShapeDtypeStruct((NC, NPAD, 128), f32),
        mesh=plsc.VectorSubcoreMesh(core_axis_name="c", subcore_axis_name="s",
                                    num_cores=NC, num_subcores=NS),
        scratch_types=[
            pltpu.VMEM((KCH, CH), jnp.int32),
            pltpu.VMEM((CH, 2 * MID), f32),
            pltpu.VMEM((CH, 2 * MID), f32),
            pltpu.VMEM((CH, 128), f32),
            pltpu.VMEM_SHARED((NPAD, 128), f32),
            pltpu.SemaphoreType.DMA,
            pltpu.SemaphoreType.DMA,
        ],
    )(_scatter_body)
    parts = scatter(idx3, c4, zeros)

    out = pl.pallas_call(
        _combine_body,
        grid=(GRID_N,),
        in_specs=[
            pl.BlockSpec((NC, BN, 128), lambda g: (0, g, 0)),
            pl.BlockSpec((BN, IN_PLANES), lambda g: (g, 0)),
        ],
        out_specs=pl.BlockSpec((BN, IN_PLANES), lambda g: (g, 0)),
        out_shape=jax.ShapeDtypeStruct((N_NODES, IN_PLANES), f32),
        compiler_params=pltpu.CompilerParams(
            dimension_semantics=("arbitrary",)),
    )(parts, x_center)
    return out
